# baseline (speedup 1.0000x reference)
import jax
import jax.numpy as jnp
from jax import lax
from jax.experimental import pallas as pl
from jax.experimental.pallas import tpu as pltpu

_PLACEHOLDER_TOKEN = 42
_B = 1024
_N = 77
_D = 768
_ROWS = _B * _N
_BIG = 2**30


def _body(tok_ref, emb_alias, ph_hbm, out_hbm, fix_sem):
    del emb_alias
    tok = tok_ref[...]
    m = tok == _PLACEHOLDER_TOKEN
    nid = lax.broadcasted_iota(jnp.int32, (_N, _B), 0)
    bid = lax.broadcasted_iota(jnp.int32, (_N, _B), 1)
    flat2 = nid * _B + bid

    def fix(carry):
        prev, _ = carry
        flat = jnp.min(jnp.where(m & (flat2 > prev), flat2, jnp.int32(_BIG)))
        has = flat < _BIG

        @pl.when(has)
        def _():
            dma = pltpu.make_async_copy(ph_hbm.at[0], out_hbm.at[flat], fix_sem)
            dma.start()
            dma.wait()

        return flat, has

    lax.while_loop(lambda c: c[1], fix, (jnp.int32(-1), True))


@jax.jit
def _scatter_overwrite(tok_nb, emb, ph):
    return pl.pallas_call(
        _body,
        grid=(),
        in_specs=[
            pl.BlockSpec(memory_space=pltpu.MemorySpace.VMEM),
            pl.BlockSpec(memory_space=pltpu.MemorySpace.HBM),
            pl.BlockSpec(memory_space=pltpu.MemorySpace.HBM),
        ],
        out_specs=pl.BlockSpec(memory_space=pltpu.MemorySpace.HBM),
        out_shape=jax.ShapeDtypeStruct((_ROWS, _D), jnp.float32),
        input_output_aliases={1: 0},
        scratch_shapes=[
            pltpu.SemaphoreType.DMA,
        ],
    )(tok_nb, emb, ph)


def kernel(reference_img, tokenized_text, embedded_text, placeholder_embedding):
    tok = tokenized_text.transpose(1, 0)
    emb = embedded_text.transpose(1, 0, 2).reshape(_ROWS, _D)
    out = _scatter_overwrite(tok, emb, placeholder_embedding)
    return out.reshape(_N, _B, _D).transpose(1, 0, 2)


# speedup vs baseline: 1.0010x; 1.0010x over previous
import jax
import jax.numpy as jnp
from jax import lax
from jax.experimental import pallas as pl
from jax.experimental.pallas import tpu as pltpu

_PLACEHOLDER_TOKEN = 42
_B = 1024
_N = 77
_D = 768
_ROWS = _B * _N
_BIG = 2**30


def _body(tok_ref, emb_alias, ph_hbm, out_hbm, fix_sem):
    del emb_alias
    tok = tok_ref[...]
    m = tok == _PLACEHOLDER_TOKEN
    nid = lax.broadcasted_iota(jnp.int32, (_N, _B), 0)
    bid = lax.broadcasted_iota(jnp.int32, (_N, _B), 1)
    flat2 = nid * _B + bid

    def fix(carry):
        prev, _ = carry
        flat = jnp.min(jnp.where(m & (flat2 > prev), flat2, jnp.int32(_BIG)))
        has = flat < _BIG

        @pl.when(has)
        def _():
            dma = pltpu.make_async_copy(ph_hbm.at[0], out_hbm.at[flat], fix_sem)
            dma.start()
            dma.wait()

        return flat, has

    lax.while_loop(lambda c: c[1], fix, (jnp.int32(-1), True))


@jax.jit
def _scatter_overwrite(tok_nb, emb, ph):
    return pl.pallas_call(
        _body,
        grid=(),
        in_specs=[
            pl.BlockSpec(memory_space=pltpu.MemorySpace.VMEM),
            pl.BlockSpec(memory_space=pltpu.MemorySpace.HBM),
            pl.BlockSpec(memory_space=pltpu.MemorySpace.HBM),
        ],
        out_specs=pl.BlockSpec(memory_space=pltpu.MemorySpace.HBM),
        out_shape=jax.ShapeDtypeStruct((_ROWS, _D), jnp.float32),
        input_output_aliases={1: 0},
        compiler_params=pltpu.CompilerParams(skip_device_barrier=True),
        scratch_shapes=[
            pltpu.SemaphoreType.DMA,
        ],
    )(tok_nb, emb, ph)


def kernel(reference_img, tokenized_text, embedded_text, placeholder_embedding):
    tok = tokenized_text.transpose(1, 0)
    emb = embedded_text.transpose(1, 0, 2).reshape(_ROWS, _D)
    out = _scatter_overwrite(tok, emb, placeholder_embedding)
    return out.reshape(_N, _B, _D).transpose(1, 0, 2)
